# baseline (speedup 1.0000x reference)
import jax
import jax.numpy as jnp
from jax.experimental import pallas as pl

MAXLEN = 200
EMBED_DIM = 128
B_BLK = 32


def _add_kernel(x_ref, pos_ref, out_ref):
    out_ref[...] = x_ref[...] + pos_ref[...][None, :, :]


def kernel(x, pos_table):
    batch = x.shape[0]
    grid = (batch // B_BLK,)
    return pl.pallas_call(
        _add_kernel,
        grid=grid,
        in_specs=[
            pl.BlockSpec((B_BLK, MAXLEN, EMBED_DIM), lambda i: (i, 0, 0)),
            pl.BlockSpec((MAXLEN, EMBED_DIM), lambda i: (0, 0)),
        ],
        out_specs=pl.BlockSpec((B_BLK, MAXLEN, EMBED_DIM), lambda i: (i, 0, 0)),
        out_shape=jax.ShapeDtypeStruct((batch, MAXLEN, EMBED_DIM), x.dtype),
    )(x, pos_table)


# speedup vs baseline: 1.0177x; 1.0177x over previous
import jax
import jax.numpy as jnp
from jax.experimental import pallas as pl

MAXLEN = 200
EMBED_DIM = 128
B_BLK = 64


def _add_kernel(x_ref, pos_ref, out_ref):
    out_ref[...] = x_ref[...] + pos_ref[...][None, :, :]


def kernel(x, pos_table):
    batch = x.shape[0]
    grid = (batch // B_BLK,)
    return pl.pallas_call(
        _add_kernel,
        grid=grid,
        in_specs=[
            pl.BlockSpec((B_BLK, MAXLEN, EMBED_DIM), lambda i: (i, 0, 0)),
            pl.BlockSpec((MAXLEN, EMBED_DIM), lambda i: (0, 0)),
        ],
        out_specs=pl.BlockSpec((B_BLK, MAXLEN, EMBED_DIM), lambda i: (i, 0, 0)),
        out_shape=jax.ShapeDtypeStruct((batch, MAXLEN, EMBED_DIM), x.dtype),
    )(x, pos_table)


# speedup vs baseline: 1.0262x; 1.0083x over previous
import jax
import jax.numpy as jnp
from jax.experimental import pallas as pl

MAXLEN = 200
EMBED_DIM = 128
B_BLK = 128


def _add_kernel(x_ref, pos_ref, out_ref):
    out_ref[...] = x_ref[...] + pos_ref[...][None, :, :]


def kernel(x, pos_table):
    batch = x.shape[0]
    grid = (batch // B_BLK,)
    return pl.pallas_call(
        _add_kernel,
        grid=grid,
        in_specs=[
            pl.BlockSpec((B_BLK, MAXLEN, EMBED_DIM), lambda i: (i, 0, 0)),
            pl.BlockSpec((MAXLEN, EMBED_DIM), lambda i: (0, 0)),
        ],
        out_specs=pl.BlockSpec((B_BLK, MAXLEN, EMBED_DIM), lambda i: (i, 0, 0)),
        out_shape=jax.ShapeDtypeStruct((batch, MAXLEN, EMBED_DIM), x.dtype),
    )(x, pos_table)
